# baseline (speedup 1.0000x reference)
import functools

import jax
import jax.numpy as jnp
from jax import lax
from jax.experimental import pallas as pl
from jax.experimental.pallas import tpu as pltpu
from jax.experimental.pallas import tpu_sc as plsc

N_BATCH = 32
N_NODE = 128
D_NODE = 512
D_EDGE = 16
BB = 2
M = BB * N_NODE
R_TOTAL = N_BATCH * N_NODE
N_WORKERS = 32
R_PER_W = R_TOTAL // N_WORKERS
G_ROWS = 8
N_GROUPS = R_PER_W // G_ROWS


def _sc_edge_body(e_hbm, a_hbm, out_hbm, e_v, a_s, out_v):
    wid = lax.axis_index("s") * 2 + lax.axis_index("c")
    base = wid * R_PER_W

    def group(g, carry):
        row0 = base + g * G_ROWS
        pltpu.sync_copy(e_hbm.at[pl.ds(row0, G_ROWS)], e_v)
        pltpu.sync_copy(a_hbm.at[pl.ds(row0, G_ROWS)], a_s)
        for rl in range(G_ROWS):
            zero = jnp.zeros((D_EDGE,), jnp.float32)
            def jblock(jc, accs):
                a0, a1, a2, a3 = accs
                jb = jc * 16
                eb = jc * 256
                a_chunk = a_s[rl, pl.ds(jb, 16)]
                for ju in range(0, 16, 4):
                    a0 = a0 + a_chunk[ju] * e_v[rl, pl.ds(eb + ju * 16, 16)]
                    a1 = a1 + a_chunk[ju + 1] * e_v[rl, pl.ds(eb + ju * 16 + 16, 16)]
                    a2 = a2 + a_chunk[ju + 2] * e_v[rl, pl.ds(eb + ju * 16 + 32, 16)]
                    a3 = a3 + a_chunk[ju + 3] * e_v[rl, pl.ds(eb + ju * 16 + 48, 16)]
                return (a0, a1, a2, a3)
            a0, a1, a2, a3 = lax.fori_loop(
                0, N_NODE // 16, jblock, (zero, zero, zero, zero))
            out_v[pl.ds((g * G_ROWS + rl) * D_EDGE, D_EDGE)] = (a0 + a1) + (a2 + a3)
        return carry

    lax.fori_loop(0, N_GROUPS, group, 0)
    pltpu.sync_copy(out_v, out_hbm.at[pl.ds(base * D_EDGE, R_PER_W * D_EDGE)])


def _sc_edge(E3, A2):
    kfn = functools.partial(
        pl.kernel,
        mesh=plsc.VectorSubcoreMesh(core_axis_name="c", subcore_axis_name="s"),
        out_type=jax.ShapeDtypeStruct((R_TOTAL * D_EDGE,), jnp.float32),
        scratch_types=[
            pltpu.VMEM((G_ROWS, N_NODE * D_EDGE), jnp.float32),
            pltpu.VMEM((G_ROWS, N_NODE), jnp.float32),
            pltpu.VMEM((R_PER_W * D_EDGE,), jnp.float32),
        ],
    )(_sc_edge_body)
    return kfn(E3, A2)


def _dense_kernel(h_ref, a_ref, wcat_ref, b_ref, p_ref):
    h = h_ref[...].reshape(M, D_NODE)
    a = a_ref[...].reshape(M, N_NODE)
    deg = jnp.sum(a, axis=1, keepdims=True)
    y = jnp.dot(h.astype(jnp.bfloat16), wcat_ref[...],
                preferred_element_type=jnp.float32)
    p = y[:, :D_NODE] + deg * y[:, D_NODE:] + b_ref[...]
    p_ref[...] = p.reshape(BB, N_NODE, D_NODE).astype(jnp.bfloat16)


def _final_kernel(p_ref, he_ref, wbot_ref, o_ref):
    he = he_ref[...].astype(jnp.bfloat16)
    out = jnp.dot(he, wbot_ref[...],
                  preferred_element_type=jnp.float32)
    out += p_ref[...].reshape(M, D_NODE).astype(jnp.float32)
    o_ref[...] = out.reshape(BB, N_NODE, D_NODE)


def kernel(H, A, E, N, W_self, W_nei, bias):
    del N
    E3 = E.reshape(R_TOTAL, N_NODE * D_EDGE)
    A2 = A.reshape(R_TOTAL, N_NODE)
    W_cat = jnp.concatenate([W_self, W_nei[:D_NODE]], axis=1).astype(jnp.bfloat16)
    W_bot = W_nei[D_NODE:].astype(jnp.bfloat16)
    bias2d = bias.reshape(1, D_NODE)

    HE = _sc_edge(E3, A2).reshape(R_TOTAL, D_EDGE)

    grid = N_BATCH // BB
    P = pl.pallas_call(
        _dense_kernel,
        grid=(grid,),
        in_specs=[
            pl.BlockSpec((BB, N_NODE, D_NODE), lambda b: (b, 0, 0)),
            pl.BlockSpec((BB, N_NODE, N_NODE), lambda b: (b, 0, 0)),
            pl.BlockSpec((D_NODE, 2 * D_NODE), lambda b: (0, 0)),
            pl.BlockSpec((1, D_NODE), lambda b: (0, 0)),
        ],
        out_specs=pl.BlockSpec((BB, N_NODE, D_NODE), lambda b: (b, 0, 0)),
        out_shape=jax.ShapeDtypeStruct((N_BATCH, N_NODE, D_NODE), jnp.bfloat16),
    )(H, A, W_cat, bias2d)

    out = pl.pallas_call(
        _final_kernel,
        grid=(grid,),
        in_specs=[
            pl.BlockSpec((BB, N_NODE, D_NODE), lambda b: (b, 0, 0)),
            pl.BlockSpec((M, D_EDGE), lambda b: (b, 0)),
            pl.BlockSpec((D_EDGE, D_NODE), lambda b: (0, 0)),
        ],
        out_specs=pl.BlockSpec((BB, N_NODE, D_NODE), lambda b: (b, 0, 0)),
        out_shape=jax.ShapeDtypeStruct((N_BATCH, N_NODE, D_NODE), jnp.float32),
    )(P, HE, W_bot)
    return out


# speedup vs baseline: 5.4450x; 5.4450x over previous
import jax
import jax.numpy as jnp
from jax.experimental import pallas as pl

N_BATCH = 32
N_NODE = 128
D_NODE = 512
D_EDGE = 16
BB = 2
M = BB * N_NODE


def _mp_kernel(h_ref, a_ref, e_ref, wcat_ref, wbot_ref, b_ref, o_ref):
    h = h_ref[...].reshape(M, D_NODE)
    a = a_ref[...].reshape(BB, N_NODE, N_NODE)

    deg = jnp.sum(a, axis=2).reshape(M, 1)
    y = jnp.dot(h.astype(jnp.bfloat16), wcat_ref[...],
                preferred_element_type=jnp.float32)
    out = y[:, :D_NODE] + deg * y[:, D_NODE:]

    t = e_ref[...] * a[:, :, None, :]
    he = jnp.sum(t, axis=3).reshape(M, D_EDGE)
    out += jnp.dot(he.astype(jnp.bfloat16), wbot_ref[...],
                   preferred_element_type=jnp.float32)

    o_ref[...] = (out + b_ref[...]).reshape(BB, N_NODE, D_NODE)


def kernel(H, A, E, N, W_self, W_nei, bias):
    del N
    E_t = jnp.transpose(E, (0, 1, 3, 2))
    W_cat = jnp.concatenate([W_self, W_nei[:D_NODE]], axis=1).astype(jnp.bfloat16)
    W_bot = W_nei[D_NODE:].astype(jnp.bfloat16)
    bias2d = bias.reshape(1, D_NODE)

    grid = N_BATCH // BB
    out = pl.pallas_call(
        _mp_kernel,
        grid=(grid,),
        in_specs=[
            pl.BlockSpec((BB, N_NODE, D_NODE), lambda b: (b, 0, 0)),
            pl.BlockSpec((BB, N_NODE, N_NODE), lambda b: (b, 0, 0)),
            pl.BlockSpec((BB, N_NODE, D_EDGE, N_NODE), lambda b: (b, 0, 0, 0)),
            pl.BlockSpec((D_NODE, 2 * D_NODE), lambda b: (0, 0)),
            pl.BlockSpec((D_EDGE, D_NODE), lambda b: (0, 0)),
            pl.BlockSpec((1, D_NODE), lambda b: (0, 0)),
        ],
        out_specs=pl.BlockSpec((BB, N_NODE, D_NODE), lambda b: (b, 0, 0)),
        out_shape=jax.ShapeDtypeStruct((N_BATCH, N_NODE, D_NODE), jnp.float32),
    )(H, A, E_t, W_cat, W_bot, bias2d)
    return out


# speedup vs baseline: 6.4746x; 1.1891x over previous
import jax
import jax.numpy as jnp
from jax.experimental import pallas as pl

N_BATCH = 32
N_NODE = 128
D_NODE = 512
D_EDGE = 16
BB = 4
M = BB * N_NODE


def _mp_kernel(h_ref, a_ref, e_ref, wcat_ref, wbot_ref, b_ref, o_ref):
    h = h_ref[...].reshape(M, D_NODE)
    a = a_ref[...].reshape(BB, N_NODE, N_NODE)

    deg = jnp.sum(a, axis=2).reshape(M, 1)
    y = jnp.dot(h.astype(jnp.bfloat16), wcat_ref[...],
                preferred_element_type=jnp.float32)
    out = y[:, :D_NODE] + deg * y[:, D_NODE:]

    t = e_ref[...] * a[:, :, None, :]
    he = jnp.sum(t, axis=3).reshape(M, D_EDGE)
    out += jnp.dot(he.astype(jnp.bfloat16), wbot_ref[...],
                   preferred_element_type=jnp.float32)

    o_ref[...] = (out + b_ref[...]).reshape(BB, N_NODE, D_NODE)


def kernel(H, A, E, N, W_self, W_nei, bias):
    del N
    E_t = jnp.transpose(E, (0, 1, 3, 2))
    W_cat = jnp.concatenate([W_self, W_nei[:D_NODE]], axis=1).astype(jnp.bfloat16)
    W_bot = W_nei[D_NODE:].astype(jnp.bfloat16)
    bias2d = bias.reshape(1, D_NODE)

    grid = N_BATCH // BB
    out = pl.pallas_call(
        _mp_kernel,
        grid=(grid,),
        in_specs=[
            pl.BlockSpec((BB, N_NODE, D_NODE), lambda b: (b, 0, 0)),
            pl.BlockSpec((BB, N_NODE, N_NODE), lambda b: (b, 0, 0)),
            pl.BlockSpec((BB, N_NODE, D_EDGE, N_NODE), lambda b: (b, 0, 0, 0)),
            pl.BlockSpec((D_NODE, 2 * D_NODE), lambda b: (0, 0)),
            pl.BlockSpec((D_EDGE, D_NODE), lambda b: (0, 0)),
            pl.BlockSpec((1, D_NODE), lambda b: (0, 0)),
        ],
        out_specs=pl.BlockSpec((BB, N_NODE, D_NODE), lambda b: (b, 0, 0)),
        out_shape=jax.ShapeDtypeStruct((N_BATCH, N_NODE, D_NODE), jnp.float32),
    )(H, A, E_t, W_cat, W_bot, bias2d)
    return out
